# Initial kernel scaffold; baseline (speedup 1.0000x reference)
import jax
import jax.numpy as jnp
from jax.experimental import pallas as pl


def kernel(x, edge_index, W_src, b_src, W_dst, b_dst):
    raise NotImplementedError("write your pallas kernel here")


# speedup vs baseline: 6.7002x; 6.7002x over previous
import jax
import jax.numpy as jnp
from jax import lax
from jax.experimental import pallas as pl
from jax.experimental.pallas import tpu as pltpu
from jax.experimental.pallas import tpu_sc as plsc

N = 10000
E = 320000
D = 128

NC, NS = 2, 16
NW = NC * NS
CHUNK = 128
CPT = 80
EPAD = NW * CPT * CHUNK
NPAD = 10240
SENT = NPAD - 1
NBR = NPAD // CHUNK
SPM_HR = 256
HR_PER_TILE = SPM_HR // NS
ROWS_PER_TILE = NPAD // NS
NPASS = 2
PASS_CH = CPT // NPASS

_MESH = plsc.VectorSubcoreMesh(
    core_axis_name="c", subcore_axis_name="s", num_cores=NC, num_subcores=NS)
_SC_PARAMS = pltpu.CompilerParams(needs_layout_passes=False)

def _zero_rows(ref, nrows):
    z16 = jnp.zeros((16,), jnp.float32)

    def body(r, _):
        for k in range(8):
            ref[r, pl.ds(k * 16, 16)] = z16
        return 0
    lax.fori_loop(0, nrows, body, 0)


def _sc_degrees(row_hbm, col_hbm, deg_hbm, rowv, colv, h):
    c = lax.axis_index("c")
    s = lax.axis_index("s")
    wid = c * NS + s
    pltpu.sync_copy(row_hbm.at[pl.ds(wid * CPT, CPT)], rowv)
    pltpu.sync_copy(col_hbm.at[pl.ds(wid * CPT, CPT)], colv)

    z16 = jnp.zeros((16,), jnp.float32)

    def zloop(r, _):
        h[pl.ds(pl.multiple_of(r * 16, 16), 16)] = z16
        return 0
    lax.fori_loop(0, 2 * NPAD // 16, zloop, 0)

    ones16 = jnp.ones((16,), jnp.float32)

    def hloop(j, _):
        for k in range(8):
            cv = colv[j, pl.ds(k * 16, 16)]
            plsc.addupdate_scatter(h, [cv], ones16)
            rv = rowv[j, pl.ds(k * 16, 16)]
            plsc.addupdate_scatter(h, [rv + NPAD], ones16)
        return 0
    lax.fori_loop(0, CPT, hloop, 0)

    pltpu.sync_copy(h, deg_hbm.at[wid])


_degrees = pl.kernel(
    _sc_degrees,
    out_type=jax.ShapeDtypeStruct((NW, 2 * NPAD), jnp.float32),
    mesh=_MESH,
    compiler_params=_SC_PARAMS,
    scratch_types=[
        pltpu.VMEM((CPT, CHUNK), jnp.int32),
        pltpu.VMEM((CPT, CHUNK), jnp.int32),
        pltpu.VMEM((2 * NPAD,), jnp.float32),
    ],
)


BLK_P = 256


def _tc_project(x_ref, w_ref, deg_ref, z_ref):
    d = jnp.sum(deg_ref[:, 0, :, :], axis=0)
    inv = jnp.where(d > 0, lax.rsqrt(d), 0.0)
    y = lax.dot_general(x_ref[...], w_ref[...], (((1,), (1,)), ((), ())),
                        preferred_element_type=jnp.float32)
    z_ref[...] = y * inv


_project = pl.pallas_call(
    _tc_project,
    grid=(NPAD // BLK_P,),
    in_specs=[
        pl.BlockSpec((BLK_P, D), lambda i: (i, 0)),
        pl.BlockSpec((D, D), lambda i: (0, 0)),
        pl.BlockSpec((NW, 1, BLK_P, 1), lambda i: (0, 0, i, 0)),
    ],
    out_specs=pl.BlockSpec((BLK_P, D), lambda i: (i, 0)),
    out_shape=jax.ShapeDtypeStruct((NPAD, D), jnp.float32),
)


def _sc_scatter(row_hbm, col_hbm, z_hbm, acc_hbm, rowv, colv, zbuf, accs,
                gsa, gsb, ssa, ssb):
    c = lax.axis_index("c")
    s = lax.axis_index("s")
    wid = c * NS + s

    _zero_rows(zbuf.at[0], CHUNK)
    for k in range(ROWS_PER_TILE // CHUNK):
        pltpu.sync_copy(zbuf.at[0],
                        accs.at[pl.ds(s * ROWS_PER_TILE + k * CHUNK, CHUNK)])
    plsc.subcore_barrier()

    def gather(j, b, sem):
        return pltpu.async_copy(z_hbm.at[colv.at[j]], zbuf.at[b], sem)

    def scatter(j, b, sem):
        return pltpu.async_copy(zbuf.at[b], accs.at[rowv.at[j]], sem, add=True)

    for p in range(NPASS):
        base = wid * CPT + p * PASS_CH
        pltpu.sync_copy(row_hbm.at[pl.ds(base, PASS_CH)], rowv)
        pltpu.sync_copy(col_hbm.at[pl.ds(base, PASS_CH)], colv)

        def body(i, _):
            ja = 2 * i
            jb = 2 * i + 1
            da = gather(ja, 0, gsa)
            db = gather(jb, 1, gsb)
            da.wait()
            sa = scatter(ja, 0, ssa)
            db.wait()
            sb = scatter(jb, 1, ssb)
            sa.wait()
            sb.wait()
            return 0
        lax.fori_loop(0, PASS_CH // 2, body, 0)

    plsc.subcore_barrier()
    pltpu.sync_copy(accs.at[pl.ds(s * ROWS_PER_TILE, ROWS_PER_TILE)],
                    acc_hbm.at[c, pl.ds(s * ROWS_PER_TILE, ROWS_PER_TILE)])


_scatter = pl.kernel(
    _sc_scatter,
    out_type=jax.ShapeDtypeStruct((NC, NPAD, D), jnp.float32),
    mesh=_MESH,
    compiler_params=_SC_PARAMS,
    scratch_types=[
        pltpu.VMEM((PASS_CH, CHUNK), jnp.int32),
        pltpu.VMEM((PASS_CH, CHUNK), jnp.int32),
        pltpu.VMEM((2, CHUNK, D), jnp.float32),
        pltpu.VMEM_SHARED((NPAD, D), jnp.float32),
        pltpu.SemaphoreType.DMA,
        pltpu.SemaphoreType.DMA,
        pltpu.SemaphoreType.DMA,
        pltpu.SemaphoreType.DMA,
    ],
)


BLK_F = 400


def _tc_finalize(acc_ref, deg_ref, b_ref, o_ref):
    d = jnp.sum(deg_ref[:, 0, :, :], axis=0)
    a = jnp.where(d > 0, lax.rsqrt(d), 0.0)
    o_ref[...] = a * (acc_ref[0] + acc_ref[1]) + b_ref[...]


_finalize = pl.pallas_call(
    _tc_finalize,
    grid=(N // BLK_F,),
    in_specs=[
        pl.BlockSpec((NC, BLK_F, D), lambda i: (0, i, 0)),
        pl.BlockSpec((NW, 1, BLK_F, 1), lambda i: (0, 1, i, 0)),
        pl.BlockSpec((1, D), lambda i: (0, 0)),
    ],
    out_specs=pl.BlockSpec((BLK_F, D), lambda i: (i, 0)),
    out_shape=jax.ShapeDtypeStruct((N, D), jnp.float32),
)


def kernel(x, edge_index, W_src, b_src, W_dst, b_dst):
    del W_dst, b_dst
    pad = EPAD - E
    sent = jnp.full((pad,), SENT, jnp.int32)
    row_p = jnp.concatenate([edge_index[0], sent]).reshape(NW * CPT, CHUNK)
    col_p = jnp.concatenate([edge_index[1], sent]).reshape(NW * CPT, CHUNK)
    x_p = jnp.pad(x, ((0, NPAD - N), (0, 0)))

    deg = _degrees(row_p, col_p)
    deg4 = deg.reshape(NW, 2, NPAD, 1)
    z = _project(x_p, W_src, deg4)
    acc = _scatter(row_p, col_p, z)
    return _finalize(acc, deg4, b_src.reshape(1, D))


# speedup vs baseline: 10.7549x; 1.6052x over previous
import jax
import jax.numpy as jnp
from jax import lax
from jax.experimental import pallas as pl
from jax.experimental.pallas import tpu as pltpu
from jax.experimental.pallas import tpu_sc as plsc

N = 10000
E = 320000
D = 128

NC, NS = 2, 16
NW = NC * NS
CHUNK = 128
CPT = 80
EPAD = NW * CPT * CHUNK
NPAD = 10240
SENT = NPAD - 1
NBR = NPAD // CHUNK
SPM_HR = 256
HR_PER_TILE = SPM_HR // NS
ROWS_PER_TILE = NPAD // NS
NPASS = 2
PASS_CH = CPT // NPASS

_MESH = plsc.VectorSubcoreMesh(
    core_axis_name="c", subcore_axis_name="s", num_cores=NC, num_subcores=NS)
_SC_PARAMS = pltpu.CompilerParams(needs_layout_passes=False)

def _zero_rows(ref, nrows):
    z16 = jnp.zeros((16,), jnp.float32)

    def body(r, _):
        for k in range(8):
            ref[r, pl.ds(k * 16, 16)] = z16
        return 0
    lax.fori_loop(0, nrows, body, 0)


def _sc_degrees(row_hbm, col_hbm, deg_hbm, rowv, colv, h):
    c = lax.axis_index("c")
    s = lax.axis_index("s")
    wid = c * NS + s
    pltpu.sync_copy(row_hbm.at[pl.ds(wid * CPT, CPT)], rowv)
    pltpu.sync_copy(col_hbm.at[pl.ds(wid * CPT, CPT)], colv)

    z16 = jnp.zeros((16,), jnp.float32)

    def zloop(r, _):
        h[pl.ds(pl.multiple_of(r * 16, 16), 16)] = z16
        return 0
    lax.fori_loop(0, 2 * NPAD // 16, zloop, 0)

    ones16 = jnp.ones((16,), jnp.float32)

    def hloop(j, _):
        for k in range(8):
            cv = colv[j, pl.ds(k * 16, 16)]
            plsc.addupdate_scatter(h, [cv], ones16)
            rv = rowv[j, pl.ds(k * 16, 16)]
            plsc.addupdate_scatter(h, [rv + NPAD], ones16)
        return 0
    lax.fori_loop(0, CPT, hloop, 0)

    pltpu.sync_copy(h, deg_hbm.at[wid])


_degrees = pl.kernel(
    _sc_degrees,
    out_type=jax.ShapeDtypeStruct((NW, 2 * NPAD), jnp.float32),
    mesh=_MESH,
    compiler_params=_SC_PARAMS,
    scratch_types=[
        pltpu.VMEM((CPT, CHUNK), jnp.int32),
        pltpu.VMEM((CPT, CHUNK), jnp.int32),
        pltpu.VMEM((2 * NPAD,), jnp.float32),
    ],
)


BLK_P = 256


def _tc_project(x_ref, w_ref, deg_ref, z_ref):
    d = jnp.sum(deg_ref[:, 0, :, :], axis=0)
    inv = jnp.where(d > 0, lax.rsqrt(d), 0.0)
    y = lax.dot_general(x_ref[...], w_ref[...], (((1,), (1,)), ((), ())),
                        preferred_element_type=jnp.float32)
    z_ref[...] = y * inv


_project = pl.pallas_call(
    _tc_project,
    grid=(NPAD // BLK_P,),
    in_specs=[
        pl.BlockSpec((BLK_P, D), lambda i: (i, 0)),
        pl.BlockSpec((D, D), lambda i: (0, 0)),
        pl.BlockSpec((NW, 1, BLK_P, 1), lambda i: (0, 0, i, 0)),
    ],
    out_specs=pl.BlockSpec((BLK_P, D), lambda i: (i, 0)),
    out_shape=jax.ShapeDtypeStruct((NPAD, D), jnp.float32),
)


def _sc_scatter(row_hbm, col_hbm, z_hbm, acc_hbm, rowv, colv, zbuf, accs,
                gsa, gsb, ssa, ssb):
    c = lax.axis_index("c")
    s = lax.axis_index("s")
    wid = c * NS + s

    _zero_rows(zbuf.at[0], CHUNK)
    for k in range(ROWS_PER_TILE // CHUNK):
        pltpu.sync_copy(zbuf.at[0],
                        accs.at[pl.ds(s * ROWS_PER_TILE + k * CHUNK, CHUNK)])
    plsc.subcore_barrier()

    def gather(j, b, sem):
        return pltpu.async_copy(z_hbm.at[colv.at[j]], zbuf.at[b], sem)

    def scatter(j, b, sem):
        return pltpu.async_copy(zbuf.at[b], accs.at[rowv.at[j]], sem, add=True)

    for p in range(NPASS):
        base = wid * CPT + p * PASS_CH
        pltpu.sync_copy(row_hbm.at[pl.ds(base, PASS_CH)], rowv)
        pltpu.sync_copy(col_hbm.at[pl.ds(base, PASS_CH)], colv)

        def body(i, _):
            ja = 2 * i
            jb = 2 * i + 1
            da = gather(ja, 0, gsa)
            db = gather(jb, 1, gsb)
            da.wait()
            sa = scatter(ja, 0, ssa)
            db.wait()
            sb = scatter(jb, 1, ssb)
            sa.wait()
            sb.wait()
            return 0
        lax.fori_loop(0, PASS_CH // 2, body, 0)

    plsc.subcore_barrier()
    pltpu.sync_copy(accs.at[pl.ds(s * ROWS_PER_TILE, ROWS_PER_TILE)],
                    acc_hbm.at[c, pl.ds(s * ROWS_PER_TILE, ROWS_PER_TILE)])


_scatter = pl.kernel(
    _sc_scatter,
    out_type=jax.ShapeDtypeStruct((NC, NPAD, D), jnp.float32),
    mesh=_MESH,
    compiler_params=_SC_PARAMS,
    scratch_types=[
        pltpu.VMEM((PASS_CH, CHUNK), jnp.int32),
        pltpu.VMEM((PASS_CH, CHUNK), jnp.int32),
        pltpu.VMEM((2, CHUNK, D), jnp.float32),
        pltpu.VMEM_SHARED((NPAD, D), jnp.float32),
        pltpu.SemaphoreType.DMA,
        pltpu.SemaphoreType.DMA,
        pltpu.SemaphoreType.DMA,
        pltpu.SemaphoreType.DMA,
    ],
)


BLK_F = 400


def _tc_finalize(acc_ref, deg_ref, b_ref, o_ref):
    d = jnp.sum(deg_ref[:, 0, :, :], axis=0)
    a = jnp.where(d > 0, lax.rsqrt(d), 0.0)
    o_ref[...] = a * (acc_ref[0] + acc_ref[1]) + b_ref[...]


_finalize = pl.pallas_call(
    _tc_finalize,
    grid=(N // BLK_F,),
    in_specs=[
        pl.BlockSpec((NC, BLK_F, D), lambda i: (0, i, 0)),
        pl.BlockSpec((NW, 1, BLK_F, 1), lambda i: (0, 1, i, 0)),
        pl.BlockSpec((1, D), lambda i: (0, 0)),
    ],
    out_specs=pl.BlockSpec((BLK_F, D), lambda i: (i, 0)),
    out_shape=jax.ShapeDtypeStruct((N, D), jnp.float32),
)


def kernel(x, edge_index, W_src, b_src, W_dst, b_dst):
    del W_dst, b_dst
    pad = EPAD - E
    sent = N + (jnp.arange(pad, dtype=jnp.int32) % (NPAD - N))
    row_p = jnp.concatenate([edge_index[0], sent]).reshape(NW * CPT, CHUNK)
    col_p = jnp.concatenate([edge_index[1], sent]).reshape(NW * CPT, CHUNK)
    x_p = jnp.pad(x, ((0, NPAD - N), (0, 0)))

    deg = _degrees(row_p, col_p)
    deg4 = deg.reshape(NW, 2, NPAD, 1)
    z = _project(x_p, W_src, deg4)
    acc = _scatter(row_p, col_p, z)
    return _finalize(acc, deg4, b_src.reshape(1, D))


# speedup vs baseline: 33.3076x; 3.0970x over previous
import jax
import jax.numpy as jnp
from jax import lax
from jax.experimental import pallas as pl
from jax.experimental.pallas import tpu as pltpu
from jax.experimental.pallas import tpu_sc as plsc

N = 10000
E = 320000
D = 128

NC, NS = 2, 16
NW = NC * NS
CHUNK = 128
CPT = 80
EPAD = NW * CPT * CHUNK
NPAD = 10240
SENT = NPAD - 1
NBR = NPAD // CHUNK
SPM_HR = 256
HR_PER_TILE = SPM_HR // NS
ROWS_PER_TILE = NPAD // NS
NPASS = 2
PASS_CH = CPT // NPASS

_MESH = plsc.VectorSubcoreMesh(
    core_axis_name="c", subcore_axis_name="s", num_cores=NC, num_subcores=NS)
_SC_PARAMS = pltpu.CompilerParams(needs_layout_passes=False)

def _zero_rows(ref, nrows):
    z16 = jnp.zeros((16,), jnp.float32)

    def body(r, _):
        for k in range(8):
            ref[r, pl.ds(k * 16, 16)] = z16
        return 0
    lax.fori_loop(0, nrows, body, 0)


def _sc_degrees(row_hbm, col_hbm, deg_hbm, rowv, colv, h):
    c = lax.axis_index("c")
    s = lax.axis_index("s")
    wid = c * NS + s
    pltpu.sync_copy(row_hbm.at[pl.ds(wid * CPT, CPT)], rowv)
    pltpu.sync_copy(col_hbm.at[pl.ds(wid * CPT, CPT)], colv)

    z16 = jnp.zeros((16,), jnp.float32)

    def zloop(r, _):
        h[pl.ds(pl.multiple_of(r * 16, 16), 16)] = z16
        return 0
    lax.fori_loop(0, 2 * NPAD // 16, zloop, 0)

    ones16 = jnp.ones((16,), jnp.float32)

    def hloop(j, _):
        for k in range(8):
            cv = colv[j, pl.ds(k * 16, 16)]
            plsc.addupdate_scatter(h, [cv], ones16)
            rv = rowv[j, pl.ds(k * 16, 16)]
            plsc.addupdate_scatter(h, [rv + NPAD], ones16)
        return 0
    lax.fori_loop(0, CPT, hloop, 0)

    pltpu.sync_copy(h, deg_hbm.at[wid])


_degrees = pl.kernel(
    _sc_degrees,
    out_type=jax.ShapeDtypeStruct((NW, 2 * NPAD), jnp.float32),
    mesh=_MESH,
    compiler_params=_SC_PARAMS,
    scratch_types=[
        pltpu.VMEM((CPT, CHUNK), jnp.int32),
        pltpu.VMEM((CPT, CHUNK), jnp.int32),
        pltpu.VMEM((2 * NPAD,), jnp.float32),
    ],
)


BLK_P = 256


def _row_scale_diag(deg_blk, blk):
    d = jnp.sum(deg_blk, axis=0)
    inv = jnp.where(d > 0, lax.rsqrt(d), 0.0)
    ir = lax.broadcasted_iota(jnp.int32, (blk, blk), 0)
    ic = lax.broadcasted_iota(jnp.int32, (blk, blk), 1)
    return jnp.where(ir == ic, jnp.broadcast_to(inv, (blk, blk)), 0.0)


def _tc_project(x_ref, w_ref, deg_ref, z_ref):
    dm = _row_scale_diag(deg_ref[:, 0, :], BLK_P)
    y = lax.dot_general(x_ref[...], w_ref[...], (((1,), (1,)), ((), ())),
                        preferred_element_type=jnp.float32)
    z_ref[...] = lax.dot_general(dm, y, (((1,), (0,)), ((), ())),
                                 preferred_element_type=jnp.float32)


_project = pl.pallas_call(
    _tc_project,
    grid=(NPAD // BLK_P,),
    in_specs=[
        pl.BlockSpec((BLK_P, D), lambda i: (i, 0)),
        pl.BlockSpec((D, D), lambda i: (0, 0)),
        pl.BlockSpec((NW, 2, BLK_P), lambda i: (0, 0, i)),
    ],
    out_specs=pl.BlockSpec((BLK_P, D), lambda i: (i, 0)),
    out_shape=jax.ShapeDtypeStruct((NPAD, D), jnp.float32),
)


def _sc_scatter(row_hbm, col_hbm, z_hbm, acc_hbm, rowv, colv, zbuf, accs,
                gsa, gsb, ssa, ssb):
    c = lax.axis_index("c")
    s = lax.axis_index("s")
    wid = c * NS + s

    _zero_rows(zbuf.at[0], CHUNK)
    for k in range(ROWS_PER_TILE // CHUNK):
        pltpu.sync_copy(zbuf.at[0],
                        accs.at[pl.ds(s * ROWS_PER_TILE + k * CHUNK, CHUNK)])
    plsc.subcore_barrier()

    def gather(j, b, sem):
        return pltpu.async_copy(z_hbm.at[colv.at[j]], zbuf.at[b], sem)

    def scatter(j, b, sem):
        return pltpu.async_copy(zbuf.at[b], accs.at[rowv.at[j]], sem, add=True)

    for p in range(NPASS):
        base = wid * CPT + p * PASS_CH
        pltpu.sync_copy(row_hbm.at[pl.ds(base, PASS_CH)], rowv)
        pltpu.sync_copy(col_hbm.at[pl.ds(base, PASS_CH)], colv)

        def body(i, _):
            ja = 2 * i
            jb = 2 * i + 1
            da = gather(ja, 0, gsa)
            db = gather(jb, 1, gsb)
            da.wait()
            sa = scatter(ja, 0, ssa)
            db.wait()
            sb = scatter(jb, 1, ssb)
            sa.wait()
            sb.wait()
            return 0
        lax.fori_loop(0, PASS_CH // 2, body, 0)

    plsc.subcore_barrier()
    pltpu.sync_copy(accs.at[pl.ds(s * ROWS_PER_TILE, ROWS_PER_TILE)],
                    acc_hbm.at[c, pl.ds(s * ROWS_PER_TILE, ROWS_PER_TILE)])


_scatter = pl.kernel(
    _sc_scatter,
    out_type=jax.ShapeDtypeStruct((NC, NPAD, D), jnp.float32),
    mesh=_MESH,
    compiler_params=_SC_PARAMS,
    scratch_types=[
        pltpu.VMEM((PASS_CH, CHUNK), jnp.int32),
        pltpu.VMEM((PASS_CH, CHUNK), jnp.int32),
        pltpu.VMEM((2, CHUNK, D), jnp.float32),
        pltpu.VMEM_SHARED((NPAD, D), jnp.float32),
        pltpu.SemaphoreType.DMA,
        pltpu.SemaphoreType.DMA,
        pltpu.SemaphoreType.DMA,
        pltpu.SemaphoreType.DMA,
    ],
)


BLK_F = 512


def _tc_finalize(acc_ref, deg_ref, b_ref, o_ref):
    dm = _row_scale_diag(deg_ref[:, 1, :], BLK_F)
    s = acc_ref[0] + acc_ref[1]
    o_ref[...] = lax.dot_general(dm, s, (((1,), (0,)), ((), ())),
                                 preferred_element_type=jnp.float32) + b_ref[...]


_finalize = pl.pallas_call(
    _tc_finalize,
    grid=(NPAD // BLK_F,),
    in_specs=[
        pl.BlockSpec((NC, BLK_F, D), lambda i: (0, i, 0)),
        pl.BlockSpec((NW, 2, BLK_F), lambda i: (0, 0, i)),
        pl.BlockSpec((1, D), lambda i: (0, 0)),
    ],
    out_specs=pl.BlockSpec((BLK_F, D), lambda i: (i, 0)),
    out_shape=jax.ShapeDtypeStruct((NPAD, D), jnp.float32),
)


def kernel(x, edge_index, W_src, b_src, W_dst, b_dst):
    del W_dst, b_dst
    pad = EPAD - E
    sent = N + (jnp.arange(pad, dtype=jnp.int32) % (NPAD - N))
    row_p = jnp.concatenate([edge_index[0], sent]).reshape(NW * CPT, CHUNK)
    col_p = jnp.concatenate([edge_index[1], sent]).reshape(NW * CPT, CHUNK)
    x_p = jnp.pad(x, ((0, NPAD - N), (0, 0)))

    deg = _degrees(row_p, col_p)
    deg3 = deg.reshape(NW, 2, NPAD)
    z = _project(x_p, W_src, deg3)
    acc = _scatter(row_p, col_p, z)
    return _finalize(acc, deg3, b_src.reshape(1, D))[:N]


# speedup vs baseline: 36.9035x; 1.1080x over previous
import jax
import jax.numpy as jnp
from jax import lax
from jax.experimental import pallas as pl
from jax.experimental.pallas import tpu as pltpu
from jax.experimental.pallas import tpu_sc as plsc

N = 10000
E = 320000
D = 128

NC, NS = 2, 16
NW = NC * NS
NPAD = 10240
NBR = NPAD // 128

CH = 80
CPT = 128
EPAD = NW * CPT * CH
NBUF = 4
PASS_CH = 32
NPASS = CPT // PASS_CH
ROWS_PER_TILE = NPAD // NS

_MESH = plsc.VectorSubcoreMesh(
    core_axis_name="c", subcore_axis_name="s", num_cores=NC, num_subcores=NS)
_SC_PARAMS = pltpu.CompilerParams(needs_layout_passes=False)


def _zero_rows(ref, nrows, ncols):
    z16 = jnp.zeros((16,), jnp.float32)

    def body(r, _):
        for k in range(ncols // 16):
            ref[r, pl.ds(k * 16, 16)] = z16
        return 0
    lax.fori_loop(0, nrows, body, 0)


def _sc_degrees(row_hbm, col_hbm, deg_hbm, rowv, colv, h):
    c = lax.axis_index("c")
    s = lax.axis_index("s")
    wid = c * NS + s
    pltpu.sync_copy(row_hbm.at[pl.ds(wid * CPT, CPT)], rowv)
    pltpu.sync_copy(col_hbm.at[pl.ds(wid * CPT, CPT)], colv)

    z16 = jnp.zeros((16,), jnp.float32)

    def zloop(r, _):
        h[pl.ds(pl.multiple_of(r * 16, 16), 16)] = z16
        return 0
    lax.fori_loop(0, 2 * NPAD // 16, zloop, 0)

    ones16 = jnp.ones((16,), jnp.float32)

    def hloop(j, _):
        for k in range(CH // 16):
            cv = colv[j, pl.ds(k * 16, 16)]
            plsc.addupdate_scatter(h, [cv], ones16)
            rv = rowv[j, pl.ds(k * 16, 16)]
            plsc.addupdate_scatter(h, [rv + NPAD], ones16)
        return 0
    lax.fori_loop(0, CPT, hloop, 0)

    pltpu.sync_copy(h, deg_hbm.at[wid])


_degrees = pl.kernel(
    _sc_degrees,
    out_type=jax.ShapeDtypeStruct((NW, 2 * NPAD), jnp.float32),
    mesh=_MESH,
    compiler_params=_SC_PARAMS,
    scratch_types=[
        pltpu.VMEM((CPT, CH), jnp.int32),
        pltpu.VMEM((CPT, CH), jnp.int32),
        pltpu.VMEM((2 * NPAD,), jnp.float32),
    ],
)


BLK_P = 256


def _row_scale_diag(deg_blk, blk):
    d = jnp.sum(deg_blk, axis=0)
    inv = jnp.where(d > 0, lax.rsqrt(d), 0.0)
    ir = lax.broadcasted_iota(jnp.int32, (blk, blk), 0)
    ic = lax.broadcasted_iota(jnp.int32, (blk, blk), 1)
    return jnp.where(ir == ic, jnp.broadcast_to(inv, (blk, blk)), 0.0)


def _tc_project(x_ref, w_ref, deg_ref, z_ref):
    dm = _row_scale_diag(deg_ref[:, 0, :], BLK_P)
    y = lax.dot_general(x_ref[...], w_ref[...], (((1,), (1,)), ((), ())),
                        preferred_element_type=jnp.float32)
    z_ref[...] = lax.dot_general(dm, y, (((1,), (0,)), ((), ())),
                                 preferred_element_type=jnp.float32)


_project = pl.pallas_call(
    _tc_project,
    grid=(NPAD // BLK_P,),
    in_specs=[
        pl.BlockSpec((BLK_P, D), lambda i: (i, 0)),
        pl.BlockSpec((D, D), lambda i: (0, 0)),
        pl.BlockSpec((NW, 2, BLK_P), lambda i: (0, 0, i)),
    ],
    out_specs=pl.BlockSpec((BLK_P, D), lambda i: (i, 0)),
    out_shape=jax.ShapeDtypeStruct((NPAD, D), jnp.float32),
)


def _sc_scatter(row_hbm, col_hbm, z_hbm, acc_hbm, rowv, colv, zbuf, accs,
                gs0, gs1, gs2, gs3, ss0, ss1, ss2, ss3):
    gs = (gs0, gs1, gs2, gs3)
    ss = (ss0, ss1, ss2, ss3)
    c = lax.axis_index("c")
    s = lax.axis_index("s")
    wid = c * NS + s

    _zero_rows(zbuf.at[0], CH, D)
    for k in range(ROWS_PER_TILE // CH):
        pltpu.sync_copy(zbuf.at[0],
                        accs.at[pl.ds(s * ROWS_PER_TILE + k * CH, CH)])
    plsc.subcore_barrier()

    def gather(j, b):
        return pltpu.async_copy(z_hbm.at[colv.at[j]], zbuf.at[b], gs[b])

    def wait_gather(j, b):
        pltpu.make_async_copy(z_hbm.at[colv.at[j]], zbuf.at[b], gs[b]).wait()

    def scatter(j, b):
        return pltpu.async_copy(zbuf.at[b], accs.at[rowv.at[j]], ss[b],
                                add=True)

    def wait_scatter(j, b):
        pltpu.make_async_copy(zbuf.at[b], accs.at[rowv.at[j]], ss[b]).wait()

    for p in range(NPASS):
        base = wid * CPT + p * PASS_CH
        pltpu.sync_copy(row_hbm.at[pl.ds(base, PASS_CH)], rowv)
        pltpu.sync_copy(col_hbm.at[pl.ds(base, PASS_CH)], colv)
        for b in range(NBUF):
            gather(b, b)

        def grp(q, _):
            j0 = q * NBUF
            for b in range(NBUF):
                wait_gather(j0 + b, b)
                scatter(j0 + b, b)
            for b in range(NBUF):
                wait_scatter(j0 + b, b)
                gather(j0 + NBUF + b, b)
            return 0
        lax.fori_loop(0, PASS_CH // NBUF - 1, grp, 0)

        j0 = PASS_CH - NBUF
        for b in range(NBUF):
            wait_gather(j0 + b, b)
            scatter(j0 + b, b)
        for b in range(NBUF):
            wait_scatter(j0 + b, b)

    plsc.subcore_barrier()
    pltpu.sync_copy(accs.at[pl.ds(s * ROWS_PER_TILE, ROWS_PER_TILE)],
                    acc_hbm.at[c, pl.ds(s * ROWS_PER_TILE, ROWS_PER_TILE)])


_scatter = pl.kernel(
    _sc_scatter,
    out_type=jax.ShapeDtypeStruct((NC, NPAD, D), jnp.float32),
    mesh=_MESH,
    compiler_params=_SC_PARAMS,
    scratch_types=[
        pltpu.VMEM((PASS_CH, CH), jnp.int32),
        pltpu.VMEM((PASS_CH, CH), jnp.int32),
        pltpu.VMEM((NBUF, CH, D), jnp.float32),
        pltpu.VMEM_SHARED((NPAD, D), jnp.float32),
        pltpu.SemaphoreType.DMA,
        pltpu.SemaphoreType.DMA,
        pltpu.SemaphoreType.DMA,
        pltpu.SemaphoreType.DMA,
        pltpu.SemaphoreType.DMA,
        pltpu.SemaphoreType.DMA,
        pltpu.SemaphoreType.DMA,
        pltpu.SemaphoreType.DMA,
    ],
)


BLK_F = 512


def _tc_finalize(acc_ref, deg_ref, b_ref, o_ref):
    dm = _row_scale_diag(deg_ref[:, 1, :], BLK_F)
    sm = acc_ref[0] + acc_ref[1]
    o_ref[...] = lax.dot_general(dm, sm, (((1,), (0,)), ((), ())),
                                 preferred_element_type=jnp.float32) + b_ref[...]


_finalize = pl.pallas_call(
    _tc_finalize,
    grid=(NPAD // BLK_F,),
    in_specs=[
        pl.BlockSpec((NC, BLK_F, D), lambda i: (0, i, 0)),
        pl.BlockSpec((NW, 2, BLK_F), lambda i: (0, 0, i)),
        pl.BlockSpec((1, D), lambda i: (0, 0)),
    ],
    out_specs=pl.BlockSpec((BLK_F, D), lambda i: (i, 0)),
    out_shape=jax.ShapeDtypeStruct((NPAD, D), jnp.float32),
)


def kernel(x, edge_index, W_src, b_src, W_dst, b_dst):
    del W_dst, b_dst
    pad = EPAD - E
    sent = N + (jnp.arange(pad, dtype=jnp.int32) % (NPAD - N))
    row_p = jnp.concatenate([edge_index[0], sent]).reshape(NW * CPT, CH)
    col_p = jnp.concatenate([edge_index[1], sent]).reshape(NW * CPT, CH)
    x_p = jnp.pad(x, ((0, NPAD - N), (0, 0)))

    deg = _degrees(row_p, col_p)
    deg3 = deg.reshape(NW, 2, NPAD)
    z = _project(x_p, W_src, deg3)
    acc = _scatter(row_p, col_p, z)
    return _finalize(acc, deg3, b_src.reshape(1, D))[:N]


# speedup vs baseline: 37.9834x; 1.0293x over previous
import jax
import jax.numpy as jnp
from jax import lax
from jax.experimental import pallas as pl
from jax.experimental.pallas import tpu as pltpu
from jax.experimental.pallas import tpu_sc as plsc

N = 10000
E = 320000
D = 128

NC, NS = 2, 16
NW = NC * NS
NPAD = 10240
NBR = NPAD // 128

CH = 80
CPT = 128
EPAD = NW * CPT * CH
NBUF = 4
PASS_CH = 32
NPASS = CPT // PASS_CH
ROWS_PER_TILE = NPAD // NS

_MESH = plsc.VectorSubcoreMesh(
    core_axis_name="c", subcore_axis_name="s", num_cores=NC, num_subcores=NS)
_SC_PARAMS = pltpu.CompilerParams(needs_layout_passes=False)


def _zero_rows(ref, nrows, ncols):
    z16 = jnp.zeros((16,), jnp.float32)

    def body(r, _):
        for k in range(ncols // 16):
            ref[r, pl.ds(k * 16, 16)] = z16
        return 0
    lax.fori_loop(0, nrows, body, 0)


def _sc_degrees(row_hbm, col_hbm, zflat_hbm, deg_hbm, rowv, colv, h):
    c = lax.axis_index("c")
    s = lax.axis_index("s")
    wid = c * NS + s
    pltpu.sync_copy(row_hbm.at[pl.ds(wid * CPT, CPT)], rowv)
    pltpu.sync_copy(col_hbm.at[pl.ds(wid * CPT, CPT)], colv)
    pltpu.sync_copy(zflat_hbm, h)

    ones16 = jnp.ones((16,), jnp.float32)

    def hloop(j, _):
        for k in range(CH // 16):
            cv = colv[j, pl.ds(k * 16, 16)]
            plsc.addupdate_scatter(h, [cv], ones16)
            rv = rowv[j, pl.ds(k * 16, 16)]
            plsc.addupdate_scatter(h, [rv + NPAD], ones16)
        return 0
    lax.fori_loop(0, CPT, hloop, 0)

    pltpu.sync_copy(h.at[pl.ds(0, NPAD)], deg_hbm.at[wid, 0])
    pltpu.sync_copy(h.at[pl.ds(NPAD, NPAD)], deg_hbm.at[wid, 1])


_degrees = pl.kernel(
    _sc_degrees,
    out_type=jax.ShapeDtypeStruct((NW, 2, NPAD), jnp.float32),
    mesh=_MESH,
    compiler_params=_SC_PARAMS,
    scratch_types=[
        pltpu.VMEM((CPT, CH), jnp.int32),
        pltpu.VMEM((CPT, CH), jnp.int32),
        pltpu.VMEM((2 * NPAD,), jnp.float32),
    ],
)


BLK_P = 256


def _row_scale_diag(eye_blk, deg_blk):
    d = jnp.sum(deg_blk, axis=0, keepdims=True)
    inv = jnp.where(d > 0, lax.rsqrt(d), 0.0)
    return eye_blk * inv


def _tc_project(x_ref, w_ref, deg_ref, eye_ref, z_ref):
    dm = _row_scale_diag(eye_ref[...], deg_ref[0, :, 0, :])
    y = lax.dot_general(x_ref[...], w_ref[...], (((1,), (1,)), ((), ())),
                        preferred_element_type=jnp.float32)
    z_ref[...] = lax.dot_general(dm, y, (((1,), (0,)), ((), ())),
                                 preferred_element_type=jnp.float32)


_project = pl.pallas_call(
    _tc_project,
    grid=(NPAD // BLK_P,),
    in_specs=[
        pl.BlockSpec((BLK_P, D), lambda i: (i, 0)),
        pl.BlockSpec((D, D), lambda i: (0, 0)),
        pl.BlockSpec((1, NW, 2, BLK_P), lambda i: (0, 0, 0, i)),
        pl.BlockSpec((BLK_P, BLK_P), lambda i: (0, 0)),
    ],
    out_specs=pl.BlockSpec((BLK_P, D), lambda i: (i, 0)),
    out_shape=jax.ShapeDtypeStruct((NPAD, D), jnp.float32),
)


def _sc_scatter(row_hbm, col_hbm, z_hbm, zrows_hbm, acc_hbm, rowv, colv,
                zbuf, accs, gs0, gs1, gs2, gs3, ss0, ss1, ss2, ss3):
    gs = (gs0, gs1, gs2, gs3)
    ss = (ss0, ss1, ss2, ss3)
    c = lax.axis_index("c")
    s = lax.axis_index("s")
    wid = c * NS + s

    pltpu.sync_copy(zrows_hbm, accs.at[pl.ds(s * ROWS_PER_TILE, ROWS_PER_TILE)])
    plsc.subcore_barrier()

    def gather(j, b):
        return pltpu.async_copy(z_hbm.at[colv.at[j]], zbuf.at[b], gs[b])

    def wait_gather(j, b):
        pltpu.make_async_copy(z_hbm.at[colv.at[j]], zbuf.at[b], gs[b]).wait()

    def scatter(j, b):
        return pltpu.async_copy(zbuf.at[b], accs.at[rowv.at[j]], ss[b],
                                add=True)

    def wait_scatter(j, b):
        pltpu.make_async_copy(zbuf.at[b], accs.at[rowv.at[j]], ss[b]).wait()

    for p in range(NPASS):
        base = wid * CPT + p * PASS_CH
        pltpu.sync_copy(row_hbm.at[pl.ds(base, PASS_CH)], rowv)
        pltpu.sync_copy(col_hbm.at[pl.ds(base, PASS_CH)], colv)
        for b in range(NBUF):
            gather(b, b)

        def grp(q, _):
            j0 = q * NBUF
            for b in range(NBUF):
                wait_gather(j0 + b, b)
                scatter(j0 + b, b)
            for b in range(NBUF):
                wait_scatter(j0 + b, b)
                gather(j0 + NBUF + b, b)
            return 0
        lax.fori_loop(0, PASS_CH // NBUF - 1, grp, 0)

        j0 = PASS_CH - NBUF
        for b in range(NBUF):
            wait_gather(j0 + b, b)
            scatter(j0 + b, b)
        for b in range(NBUF):
            wait_scatter(j0 + b, b)

    plsc.subcore_barrier()
    pltpu.sync_copy(accs.at[pl.ds(s * ROWS_PER_TILE, ROWS_PER_TILE)],
                    acc_hbm.at[c, pl.ds(s * ROWS_PER_TILE, ROWS_PER_TILE)])


_scatter = pl.kernel(
    _sc_scatter,
    out_type=jax.ShapeDtypeStruct((NC, NPAD, D), jnp.float32),
    mesh=_MESH,
    compiler_params=_SC_PARAMS,
    scratch_types=[
        pltpu.VMEM((PASS_CH, CH), jnp.int32),
        pltpu.VMEM((PASS_CH, CH), jnp.int32),
        pltpu.VMEM((NBUF, CH, D), jnp.float32),
        pltpu.VMEM_SHARED((NPAD, D), jnp.float32),
        pltpu.SemaphoreType.DMA,
        pltpu.SemaphoreType.DMA,
        pltpu.SemaphoreType.DMA,
        pltpu.SemaphoreType.DMA,
        pltpu.SemaphoreType.DMA,
        pltpu.SemaphoreType.DMA,
        pltpu.SemaphoreType.DMA,
        pltpu.SemaphoreType.DMA,
    ],
)


BLK_F = 512


def _tc_finalize(acc_ref, deg_ref, eye_ref, b_ref, o_ref):
    dm = _row_scale_diag(eye_ref[...], deg_ref[0, :, 1, :])
    sm = acc_ref[0] + acc_ref[1]
    o_ref[...] = lax.dot_general(dm, sm, (((1,), (0,)), ((), ())),
                                 preferred_element_type=jnp.float32) + b_ref[...]


_finalize = pl.pallas_call(
    _tc_finalize,
    grid=(NPAD // BLK_F,),
    in_specs=[
        pl.BlockSpec((NC, BLK_F, D), lambda i: (0, i, 0)),
        pl.BlockSpec((1, NW, 2, BLK_F), lambda i: (0, 0, 0, i)),
        pl.BlockSpec((BLK_F, BLK_F), lambda i: (0, 0)),
        pl.BlockSpec((1, D), lambda i: (0, 0)),
    ],
    out_specs=pl.BlockSpec((BLK_F, D), lambda i: (i, 0)),
    out_shape=jax.ShapeDtypeStruct((N, D), jnp.float32),
)


def kernel(x, edge_index, W_src, b_src, W_dst, b_dst):
    del W_dst, b_dst
    pad = EPAD - E
    sent = N + (jnp.arange(pad, dtype=jnp.int32) % (NPAD - N))
    row_p = jnp.concatenate([edge_index[0], sent]).reshape(NW * CPT, CH)
    col_p = jnp.concatenate([edge_index[1], sent]).reshape(NW * CPT, CH)
    x_p = jnp.pad(x, ((0, NPAD - N), (0, 0)))

    zflat = jnp.zeros((2 * NPAD,), jnp.float32)
    zrows = jnp.zeros((ROWS_PER_TILE, D), jnp.float32)
    eye_p = jnp.eye(BLK_P, dtype=jnp.float32)
    eye_f = jnp.eye(BLK_F, dtype=jnp.float32)

    deg = _degrees(row_p, col_p, zflat)
    deg4 = deg[None]
    z = _project(x_p, W_src, deg4, eye_p)
    acc = _scatter(row_p, col_p, z, zrows)
    return _finalize(acc, deg4, eye_f, b_src.reshape(1, D))


# speedup vs baseline: 38.3563x; 1.0098x over previous
import jax
import jax.numpy as jnp
from jax import lax
from jax.experimental import pallas as pl
from jax.experimental.pallas import tpu as pltpu
from jax.experimental.pallas import tpu_sc as plsc

N = 10000
E = 320000
D = 128

NC, NS = 2, 16
NW = NC * NS
NPAD = 10240
NBR = NPAD // 128

CH = 80
CPT = 128
EPAD = NW * CPT * CH
NBUF = 4
PASS_CH = 32
NPASS = CPT // PASS_CH
ROWS_PER_TILE = NPAD // NS

_MESH = plsc.VectorSubcoreMesh(
    core_axis_name="c", subcore_axis_name="s", num_cores=NC, num_subcores=NS)
_SC_PARAMS = pltpu.CompilerParams(needs_layout_passes=False)


def _zero_rows(ref, nrows, ncols):
    z16 = jnp.zeros((16,), jnp.float32)

    def body(r, _):
        for k in range(ncols // 16):
            ref[r, pl.ds(k * 16, 16)] = z16
        return 0
    lax.fori_loop(0, nrows, body, 0)


def _sc_degrees(row_hbm, col_hbm, zflat_hbm, deg_hbm, rowv, colv, h):
    c = lax.axis_index("c")
    s = lax.axis_index("s")
    wid = c * NS + s
    pltpu.sync_copy(row_hbm.at[pl.ds(wid * CPT, CPT)], rowv)
    pltpu.sync_copy(col_hbm.at[pl.ds(wid * CPT, CPT)], colv)
    pltpu.sync_copy(zflat_hbm, h)

    ones16 = jnp.ones((16,), jnp.float32)

    def hloop(j, _):
        for k in range(CH // 16):
            cv = colv[j, pl.ds(k * 16, 16)]
            plsc.addupdate_scatter(h, [cv], ones16)
            rv = rowv[j, pl.ds(k * 16, 16)]
            plsc.addupdate_scatter(h, [rv + NPAD], ones16)
        return 0
    lax.fori_loop(0, CPT, hloop, 0)

    pltpu.sync_copy(h.at[pl.ds(0, NPAD)], deg_hbm.at[wid, 0])
    pltpu.sync_copy(h.at[pl.ds(NPAD, NPAD)], deg_hbm.at[wid, 1])


_degrees = pl.kernel(
    _sc_degrees,
    out_type=jax.ShapeDtypeStruct((NW, 2, NPAD), jnp.float32),
    mesh=_MESH,
    compiler_params=_SC_PARAMS,
    scratch_types=[
        pltpu.VMEM((CPT, CH), jnp.int32),
        pltpu.VMEM((CPT, CH), jnp.int32),
        pltpu.VMEM((2 * NPAD,), jnp.float32),
    ],
)


BLK_P = 256


def _row_scale_col(deg_blk):
    d = jnp.sum(deg_blk, axis=0, keepdims=True)
    inv = jnp.where(d > 0, lax.rsqrt(d), 0.0)
    return jnp.transpose(inv)


def _tc_project(x_ref, w_ref, deg_ref, z_ref):
    inv = _row_scale_col(deg_ref[0, :, 0, :])
    y = lax.dot_general(x_ref[...], w_ref[...], (((1,), (1,)), ((), ())),
                        preferred_element_type=jnp.float32)
    z_ref[...] = y * inv


_project = pl.pallas_call(
    _tc_project,
    grid=(NPAD // BLK_P,),
    in_specs=[
        pl.BlockSpec((BLK_P, D), lambda i: (i, 0)),
        pl.BlockSpec((D, D), lambda i: (0, 0)),
        pl.BlockSpec((1, NW, 2, BLK_P), lambda i: (0, 0, 0, i)),
    ],
    out_specs=pl.BlockSpec((BLK_P, D), lambda i: (i, 0)),
    out_shape=jax.ShapeDtypeStruct((NPAD, D), jnp.float32),
)


def _sc_scatter(row_hbm, col_hbm, z_hbm, zrows_hbm, acc_hbm, rowv, colv,
                zbuf, accs, gs0, gs1, gs2, gs3, ss0, ss1, ss2, ss3):
    gs = (gs0, gs1, gs2, gs3)
    ss = (ss0, ss1, ss2, ss3)
    c = lax.axis_index("c")
    s = lax.axis_index("s")
    wid = c * NS + s

    pltpu.sync_copy(zrows_hbm, accs.at[pl.ds(s * ROWS_PER_TILE, ROWS_PER_TILE)])
    plsc.subcore_barrier()

    def gather(j, b):
        return pltpu.async_copy(z_hbm.at[colv.at[j]], zbuf.at[b], gs[b])

    def wait_gather(j, b):
        pltpu.make_async_copy(z_hbm.at[colv.at[j]], zbuf.at[b], gs[b]).wait()

    def scatter(j, b):
        return pltpu.async_copy(zbuf.at[b], accs.at[rowv.at[j]], ss[b],
                                add=True)

    def wait_scatter(j, b):
        pltpu.make_async_copy(zbuf.at[b], accs.at[rowv.at[j]], ss[b]).wait()

    for p in range(NPASS):
        base = wid * CPT + p * PASS_CH
        pltpu.sync_copy(row_hbm.at[pl.ds(base, PASS_CH)], rowv)
        pltpu.sync_copy(col_hbm.at[pl.ds(base, PASS_CH)], colv)
        for b in range(NBUF):
            gather(b, b)

        def grp(q, _):
            j0 = q * NBUF
            for b in range(NBUF):
                wait_gather(j0 + b, b)
                scatter(j0 + b, b)
            for b in range(NBUF):
                wait_scatter(j0 + b, b)
                gather(j0 + NBUF + b, b)
            return 0
        lax.fori_loop(0, PASS_CH // NBUF - 1, grp, 0)

        j0 = PASS_CH - NBUF
        for b in range(NBUF):
            wait_gather(j0 + b, b)
            scatter(j0 + b, b)
        for b in range(NBUF):
            wait_scatter(j0 + b, b)

    plsc.subcore_barrier()
    pltpu.sync_copy(accs.at[pl.ds(s * ROWS_PER_TILE, ROWS_PER_TILE)],
                    acc_hbm.at[c, pl.ds(s * ROWS_PER_TILE, ROWS_PER_TILE)])


_scatter = pl.kernel(
    _sc_scatter,
    out_type=jax.ShapeDtypeStruct((NC, NPAD, D), jnp.float32),
    mesh=_MESH,
    compiler_params=_SC_PARAMS,
    scratch_types=[
        pltpu.VMEM((PASS_CH, CH), jnp.int32),
        pltpu.VMEM((PASS_CH, CH), jnp.int32),
        pltpu.VMEM((NBUF, CH, D), jnp.float32),
        pltpu.VMEM_SHARED((NPAD, D), jnp.float32),
        pltpu.SemaphoreType.DMA,
        pltpu.SemaphoreType.DMA,
        pltpu.SemaphoreType.DMA,
        pltpu.SemaphoreType.DMA,
        pltpu.SemaphoreType.DMA,
        pltpu.SemaphoreType.DMA,
        pltpu.SemaphoreType.DMA,
        pltpu.SemaphoreType.DMA,
    ],
)


BLK_F = 512


def _tc_finalize(acc_ref, deg_ref, b_ref, o_ref):
    inv = _row_scale_col(deg_ref[0, :, 1, :])
    o_ref[...] = inv * (acc_ref[0] + acc_ref[1]) + b_ref[...]


_finalize = pl.pallas_call(
    _tc_finalize,
    grid=(NPAD // BLK_F,),
    in_specs=[
        pl.BlockSpec((NC, BLK_F, D), lambda i: (0, i, 0)),
        pl.BlockSpec((1, NW, 2, BLK_F), lambda i: (0, 0, 0, i)),
        pl.BlockSpec((1, D), lambda i: (0, 0)),
    ],
    out_specs=pl.BlockSpec((BLK_F, D), lambda i: (i, 0)),
    out_shape=jax.ShapeDtypeStruct((N, D), jnp.float32),
)


def kernel(x, edge_index, W_src, b_src, W_dst, b_dst):
    del W_dst, b_dst
    pad = EPAD - E
    sent = N + (jnp.arange(pad, dtype=jnp.int32) % (NPAD - N))
    row_p = jnp.concatenate([edge_index[0], sent]).reshape(NW * CPT, CH)
    col_p = jnp.concatenate([edge_index[1], sent]).reshape(NW * CPT, CH)
    x_p = jnp.pad(x, ((0, NPAD - N), (0, 0)))

    zflat = jnp.zeros((2 * NPAD,), jnp.float32)
    zrows = jnp.zeros((ROWS_PER_TILE, D), jnp.float32)

    deg = _degrees(row_p, col_p, zflat)
    deg4 = deg[None]
    z = _project(x_p, W_src, deg4)
    acc = _scatter(row_p, col_p, z, zrows)
    return _finalize(acc, deg4, b_src.reshape(1, D))
